# baseline (speedup 1.0000x reference)
import jax
import jax.numpy as jnp
from jax.experimental import pallas as pl
from jax.experimental.pallas import tpu as pltpu
from jax.experimental.pallas import tpu_sc as plsc

T = 2048
REPR = 2048
FEAT = 1024
HID = 1024
GATE = 256
MOEH = 512
E = 16
K = 2
ACT = 12

BT = 128
NTB = T // BT
A = T * K
BM = 128
NB = A // BM
NPAIR = NB + E - 1

_F32 = jnp.float32
_I32 = jnp.int32


def _gate_kernel(obs_ref, wt_ref, bt_ref, lng_ref, lnb_ref, wp1_ref, bp1_ref,
                 wg1_ref, bg1_ref, wg2_ref, bg2_ref,
                 x_ref, topi_ref, topv_ref, rank_ref, counts_ref, offs_ref,
                 aux_ref, acc_ref):
    i = pl.program_id(0)

    @pl.when(i == 0)
    def _():
        acc_ref[...] = jnp.zeros_like(acc_ref)

    h = jnp.dot(obs_ref[...], wt_ref[...], preferred_element_type=_F32)
    h = h + bt_ref[...]
    mu_ = jnp.mean(h, axis=-1, keepdims=True)
    var = jnp.mean((h - mu_) ** 2, axis=-1, keepdims=True)
    h = (h - mu_) / jnp.sqrt(var + 1e-5) * lng_ref[...] + lnb_ref[...]
    h = jnp.tanh(h)

    x = jax.nn.relu(jnp.dot(h, wp1_ref[...], preferred_element_type=_F32)
                    + bp1_ref[...])
    x_ref[...] = x

    gh = jax.nn.relu(jnp.dot(x, wg1_ref[...], preferred_element_type=_F32)
                     + bg1_ref[...])
    logits = jnp.dot(gh, wg2_ref[...], preferred_element_type=_F32) + bg2_ref[...]
    m = jnp.max(logits, axis=-1, keepdims=True)
    p = jnp.exp(logits - m)
    probs = p / jnp.sum(p, axis=-1, keepdims=True)

    iota_e = jax.lax.broadcasted_iota(_I32, (BT, E), 1).astype(_F32)
    v1 = jnp.max(probs, axis=-1, keepdims=True)
    i1 = jnp.min(jnp.where(probs == v1, iota_e, _F32(E)), axis=-1,
                 keepdims=True)
    oh1 = (iota_e == i1)
    masked = jnp.where(oh1, -jnp.inf, probs)
    v2 = jnp.max(masked, axis=-1, keepdims=True)
    i2 = jnp.min(jnp.where(masked == v2, iota_e, _F32(E)), axis=-1,
                 keepdims=True)
    oh2 = (iota_e == i2)
    s = v1 + v2 + 1e-9
    w1 = v1 / s
    w2 = v2 / s

    oh1f = oh1.astype(_F32)
    oh2f = oh2.astype(_F32)
    tri = (jax.lax.broadcasted_iota(_I32, (BT, BT), 1)
           < jax.lax.broadcasted_iota(_I32, (BT, BT), 0)).astype(_F32)
    cum0 = jnp.dot(tri, oh1f, preferred_element_type=_F32)
    cum1 = jnp.dot(tri, oh2f, preferred_element_type=_F32)
    colsum0 = jnp.sum(oh1f, axis=0, keepdims=True)
    colsum1 = jnp.sum(oh2f, axis=0, keepdims=True)
    run = acc_ref[0:1, :]
    rank0 = jnp.sum(oh1f * (cum0 + run), axis=-1, keepdims=True)
    rank1 = jnp.sum(oh2f * (cum1 + colsum0 + run), axis=-1, keepdims=True)

    topi_ref[...] = jnp.concatenate([i1, i2], axis=1).astype(_I32)
    topv_ref[...] = jnp.concatenate([w1, w2], axis=1)
    rank_ref[...] = jnp.concatenate([rank0, rank1], axis=1).astype(_I32)

    acc_ref[0:1, :] = run + colsum0 + colsum1
    acc_ref[1:2, :] += jnp.sum(probs, axis=0, keepdims=True)
    gz = (oh1f * (w1 > 0).astype(_F32) + oh2f * (w2 > 0).astype(_F32))
    acc_ref[2:3, :] += jnp.sum(gz, axis=0, keepdims=True)

    @pl.when(i == NTB - 1)
    def _():
        cnt = acc_ref[0:1, :]
        counts_ref[...] = cnt.astype(_I32)
        excl = (jax.lax.broadcasted_iota(_I32, (E, E), 0)
                < jax.lax.broadcasted_iota(_I32, (E, E), 1)).astype(_F32)
        offs_ref[...] = jnp.dot(cnt, excl,
                                preferred_element_type=_F32).astype(_I32)
        imp = acc_ref[1:2, :] * _F32(1.0 / T)
        load = acc_ref[2:3, :] * _F32(1.0 / T)
        aux_ref[...] = jnp.sum(imp * load).reshape(1, 1) * _F32(E)


def _gate_call(obs, Wt, bt, ln_g, ln_b, Wp1, bp1, Wg1, bg1, Wg2, bg2):
    full = lambda i: (0, 0)
    blk = lambda i: (i, 0)
    return pl.pallas_call(
        _gate_kernel,
        grid=(NTB,),
        in_specs=[
            pl.BlockSpec((BT, REPR), blk),
            pl.BlockSpec((REPR, FEAT), full),
            pl.BlockSpec((1, FEAT), full),
            pl.BlockSpec((1, FEAT), full),
            pl.BlockSpec((1, FEAT), full),
            pl.BlockSpec((FEAT, HID), full),
            pl.BlockSpec((1, HID), full),
            pl.BlockSpec((HID, GATE), full),
            pl.BlockSpec((1, GATE), full),
            pl.BlockSpec((GATE, E), full),
            pl.BlockSpec((1, E), full),
        ],
        out_specs=[
            pl.BlockSpec((BT, HID), blk),
            pl.BlockSpec((BT, K), blk),
            pl.BlockSpec((BT, K), blk),
            pl.BlockSpec((BT, K), blk),
            pl.BlockSpec((1, E), full),
            pl.BlockSpec((1, E), full),
            pl.BlockSpec((1, 1), full),
        ],
        out_shape=[
            jax.ShapeDtypeStruct((T, HID), _F32),
            jax.ShapeDtypeStruct((T, K), _I32),
            jax.ShapeDtypeStruct((T, K), _F32),
            jax.ShapeDtypeStruct((T, K), _I32),
            jax.ShapeDtypeStruct((1, E), _I32),
            jax.ShapeDtypeStruct((1, E), _I32),
            jax.ShapeDtypeStruct((1, 1), _F32),
        ],
        scratch_shapes=[pltpu.VMEM((4, E), _F32)],
    )(obs, Wt, bt.reshape(1, FEAT), ln_g.reshape(1, FEAT),
      ln_b.reshape(1, FEAT), Wp1, bp1.reshape(1, HID), Wg1,
      bg1.reshape(1, GATE), Wg2, bg2.reshape(1, E))


def _expert_kernel(pb_ref, pe_ref, plo_ref, phi_ref, pf_ref,
                   xs_ref, we1_ref, be1_ref, we2_ref, be2_ref, ys_ref):
    p = pl.program_id(0)
    gr = pb_ref[p] * BM + jax.lax.broadcasted_iota(_I32, (BM, 1), 0)
    mask = (gr >= plo_ref[p]) & (gr < phi_ref[p])
    h = jax.nn.relu(jnp.dot(xs_ref[...], we1_ref[0],
                            preferred_element_type=_F32) + be1_ref[0])
    o = jnp.dot(h, we2_ref[0], preferred_element_type=_F32) + be2_ref[0]
    contrib = jnp.where(mask, o, _F32(0.0))

    @pl.when(pf_ref[p] == 1)
    def _():
        ys_ref[...] = contrib

    @pl.when(pf_ref[p] == 0)
    def _():
        ys_ref[...] += contrib


def _expert_call(pb, pe, plo, phi, pf, xs, We1, be1, We2, be2):
    grid_spec = pltpu.PrefetchScalarGridSpec(
        num_scalar_prefetch=5,
        grid=(NPAIR,),
        in_specs=[
            pl.BlockSpec((BM, HID), lambda p, pb, pe, lo, hi, f: (pb[p], 0)),
            pl.BlockSpec((1, HID, MOEH),
                         lambda p, pb, pe, lo, hi, f: (pe[p], 0, 0)),
            pl.BlockSpec((1, 1, MOEH),
                         lambda p, pb, pe, lo, hi, f: (pe[p], 0, 0)),
            pl.BlockSpec((1, MOEH, HID),
                         lambda p, pb, pe, lo, hi, f: (pe[p], 0, 0)),
            pl.BlockSpec((1, 1, HID),
                         lambda p, pb, pe, lo, hi, f: (pe[p], 0, 0)),
        ],
        out_specs=pl.BlockSpec((BM, HID),
                               lambda p, pb, pe, lo, hi, f: (pb[p], 0)),
    )
    return pl.pallas_call(
        _expert_kernel,
        grid_spec=grid_spec,
        out_shape=jax.ShapeDtypeStruct((A, HID), _F32),
    )(pb, pe, plo, phi, pf, xs, We1, be1.reshape(E, 1, MOEH), We2,
      be2.reshape(E, 1, HID))


def _combine_kernel(g0_ref, g1_ref, topv_ref, wp2_ref, bp2_ref, mu_ref):
    y = (topv_ref[:, 0:1] * g0_ref[...] + topv_ref[:, 1:2] * g1_ref[...])
    y = jax.nn.relu(y)
    mu_ref[...] = jnp.tanh(jnp.dot(y, wp2_ref[...], preferred_element_type=_F32)
                           + bp2_ref[...])


def _combine_call(g, topv, Wp2, bp2):
    return pl.pallas_call(
        _combine_kernel,
        grid=(NTB,),
        in_specs=[
            pl.BlockSpec((BT, HID), lambda i: (i, 0)),
            pl.BlockSpec((BT, HID), lambda i: (NTB + i, 0)),
            pl.BlockSpec((BT, K), lambda i: (i, 0)),
            pl.BlockSpec((HID, ACT), lambda i: (0, 0)),
            pl.BlockSpec((1, ACT), lambda i: (0, 0)),
        ],
        out_specs=pl.BlockSpec((BT, ACT), lambda i: (i, 0)),
        out_shape=jax.ShapeDtypeStruct((T, ACT), _F32),
    )(g, g, topv, Wp2, bp2.reshape(1, ACT))


def _inv_kernel(slot_ref, inv_ref, acc_ref):
    i = pl.program_id(0)

    @pl.when(i == 0)
    def _():
        acc_ref[...] = jnp.zeros_like(acc_ref)

    iota_a = jax.lax.broadcasted_iota(_I32, (BT, A), 1)
    t = i * BT + jax.lax.broadcasted_iota(_I32, (BT, 1), 0)
    oh0 = (iota_a == slot_ref[:, 0:1]).astype(_F32)
    oh1 = (iota_a == slot_ref[:, 1:2]).astype(_F32)
    acc_ref[...] += jnp.sum((oh0 + oh1) * t.astype(_F32), axis=0,
                            keepdims=True)

    @pl.when(i == NTB - 1)
    def _():
        inv_ref[...] = acc_ref[...].astype(_I32)


def _inv_call(slot):
    return pl.pallas_call(
        _inv_kernel,
        grid=(NTB,),
        in_specs=[pl.BlockSpec((BT, K), lambda i: (i, 0))],
        out_specs=pl.BlockSpec((1, A), lambda i: (0, 0)),
        out_shape=jax.ShapeDtypeStruct((1, A), _I32),
        scratch_shapes=[pltpu.VMEM((1, A), _F32)],
    )(slot)


_SC_NC = 2
_SC_NS = 16
NW = _SC_NC * _SC_NS
APW = A // NW
CH = 64
NCH = APW // CH


def _vec_mesh():
    return plsc.VectorSubcoreMesh(core_axis_name="core",
                                  subcore_axis_name="subcore")


def _sc_gather_rows(table, idx):
    return jnp.take(table, idx, axis=0)
    w = table.shape[1]

    @pl.kernel(out_type=jax.ShapeDtypeStruct((A, w), table.dtype),
               mesh=_vec_mesh(),
               scratch_types=[pltpu.VMEM((CH,), _I32),
                              pltpu.VMEM((CH, w), table.dtype),
                              pltpu.SemaphoreType.DMA])
    def k(t_hbm, i_hbm, o_hbm, i_v, r_v, sem):
        wid = (jax.lax.axis_index("subcore") * _SC_NC
               + jax.lax.axis_index("core"))
        for c in range(NCH):
            base = wid * APW + c * CH
            pltpu.sync_copy(i_hbm.at[pl.ds(base, CH)], i_v)
            pltpu.async_copy(t_hbm.at[i_v], r_v, sem).wait()
            pltpu.sync_copy(r_v, o_hbm.at[pl.ds(base, CH)])

    return k(table, idx)


def _pair_metadata(counts, offs):
    c = counts.astype(_I32)
    o = offs.astype(_I32)
    b0 = o // BM
    b1 = jnp.where(c > 0, (o + c - 1) // BM, b0)
    nblk = jnp.where(c > 0, b1 - b0 + 1, 0)
    start = jnp.cumsum(nblk) - nblk
    total = jnp.sum(nblk)
    p_ar = jnp.arange(NPAIR, dtype=_I32)
    e_p = (jnp.sum((p_ar[:, None] >= start[None, :]).astype(_I32), axis=1) - 1)
    e_p = jnp.clip(e_p, 0, E - 1)
    j = p_ar - start[e_p]
    b_p = b0[e_p] + j
    lo = jnp.maximum(o[e_p], b_p * BM)
    hi = jnp.minimum(o[e_p] + c[e_p], (b_p + 1) * BM)
    valid = p_ar < total
    b_p = jnp.where(valid, b_p, NB - 1)
    lo = jnp.where(valid, lo, 0)
    hi = jnp.where(valid, hi, 0)
    e_p = jnp.where(valid, e_p, 0)
    first = jnp.concatenate(
        [jnp.ones((1,), _I32), (b_p[1:] != b_p[:-1]).astype(_I32)])
    return b_p, e_p, lo, hi, first


def kernel(obs, std, Wt, bt, ln_g, ln_b, Wp1, bp1, Wg1, bg1, Wg2, bg2,
           We1, be1, We2, be2, Wp2, bp2):
    x, topi, topv, rank, counts, offs, aux = _gate_call(
        obs, Wt, bt, ln_g, ln_b, Wp1, bp1, Wg1, bg1, Wg2, bg2)
    counts = counts[0]
    offs = offs[0]

    slot = jnp.take(offs, topi, axis=0) + rank
    slot_km = slot.T.reshape(A)

    tok = _inv_call(slot).reshape(A)

    xs = _sc_gather_rows(x, tok)

    pb, pe, plo, phi, pf = _pair_metadata(counts, offs)
    ys = _expert_call(pb, pe, plo, phi, pf, xs, We1, be1, We2, be2)

    g = _sc_gather_rows(ys, slot_km)
    mu = _combine_call(g, topv, Wp2, bp2)

    std_t = jnp.ones_like(mu) * std
    return (mu, std_t, aux[0, 0])
